# baseline (speedup 1.0000x reference)
import functools

import jax
import jax.numpy as jnp
from jax import lax
from jax.experimental import pallas as pl
from jax.experimental.pallas import tpu as pltpu
from jax.experimental.pallas import tpu_sc as plsc

N = 10000
D = 128
HD = D // 2
E = 320000
OWNER = 1.0

NC = 2
NS = 16
NW = NC * NS
CHUNK = 120
DCHUNK = 128
DCPT = -(-E // (NW * DCHUNK))
DEPAD = DCPT * DCHUNK * NW
FAST_CID = 0
CPT_SLOW = 76
CPT_FAST = 92
DUMMY = N
ACC_ROWS = 10112
ZROWS = ACC_ROWS // NS

_mesh = plsc.VectorSubcoreMesh(core_axis_name="c", subcore_axis_name="s",
                               num_cores=NC, num_subcores=NS)
_sc_params = pltpu.CompilerParams(
    needs_layout_passes=False, use_tc_tiling_on_sc=False)
_f32 = jnp.float32


def _deg_body(dst_hbm, out_hbm, dst_v, hist_v):
    wid = lax.axis_index("c") * NS + lax.axis_index("s")
    pltpu.sync_copy(dst_hbm.at[wid], dst_v)
    zeros = jnp.zeros((16,), _f32)

    def zbody(i, c):
        hist_v[pl.ds(i * 16, 16)] = zeros
        return c

    lax.fori_loop(0, ACC_ROWS // 16, zbody, 0)
    ones = jnp.ones((16,), _f32)

    def body(j, c):
        for k in range(DCHUNK // 16):
            idx = dst_v[j, pl.ds(k * 16, 16)]
            plsc.addupdate_scatter(hist_v, [idx], ones)
        return c

    lax.fori_loop(0, DCPT, body, 0)
    pltpu.sync_copy(hist_v, out_hbm.at[pl.ds(wid * ACC_ROWS, ACC_ROWS)])


def _prop_body(gA, gB, src_hbm, dst_hbm, outA, outB,
               src_v, dst_v, b0, acc, g0):
    cid = lax.axis_index("c")
    sid = lax.axis_index("s")
    wid = cid * NS + sid
    pltpu.sync_copy(src_hbm.at[wid], src_v)
    pltpu.sync_copy(dst_hbm.at[wid], dst_v)
    zeros = jnp.zeros((16,), _f32)
    zbase = sid * ZROWS
    nfull = ZROWS // CHUNK
    rem = ZROWS - nfull * CHUNK

    for p in range(2):
        g_hbm = gA if p == 0 else gB
        out_hbm = outA if p == 0 else outB

        def zrow(i, c):
            for k in range(HD // 16):
                b0[i, pl.ds(k * 16, 16)] = zeros
            return c

        lax.fori_loop(0, CHUNK, zrow, 0)
        for c in range(nfull):
            pltpu.sync_copy(b0, acc.at[pl.ds(zbase + c * CHUNK, CHUNK)])
        if rem:
            pltpu.sync_copy(b0.at[pl.ds(0, rem)],
                            acc.at[pl.ds(zbase + nfull * CHUNK, rem)])
        plsc.subcore_barrier()

        def body(j, c):
            desc = pltpu.make_async_copy(g_hbm.at[src_v.at[j]], b0, g0)
            desc.start()
            desc.wait()
            pltpu.sync_copy(b0, acc.at[dst_v.at[j]], add=True)
            return c

        lax.fori_loop(0, CPT_SLOW, body, 0)

        @pl.when(cid == FAST_CID)
        def _rest():
            lax.fori_loop(CPT_SLOW, CPT_FAST, body, 0)
        plsc.subcore_barrier()
        pltpu.sync_copy(acc.at[pl.ds(zbase, ZROWS)],
                        out_hbm.at[pl.ds(cid * ACC_ROWS + zbase, ZROWS)])
        if p == 0:
            plsc.subcore_barrier()


def _make_deg(interpret=False):
    return pl.kernel(
        _deg_body,
        out_type=jax.ShapeDtypeStruct((NW * ACC_ROWS,), _f32),
        mesh=_mesh,
        compiler_params=_sc_params,
        scratch_types=[
            pltpu.VMEM((DCPT, DCHUNK), jnp.int32),
            pltpu.VMEM((ACC_ROWS,), _f32),
        ],
        interpret=interpret,
    )


def _make_prop(interpret=False):
    return pl.kernel(
        _prop_body,
        out_type=(jax.ShapeDtypeStruct((NC * ACC_ROWS, HD), _f32),
                  jax.ShapeDtypeStruct((NC * ACC_ROWS, HD), _f32)),
        mesh=_mesh,
        compiler_params=_sc_params,
        scratch_types=[
            pltpu.VMEM((CPT_FAST, CHUNK), jnp.int32),
            pltpu.VMEM((CPT_FAST, CHUNK), jnp.int32),
            pltpu.VMEM((CHUNK, HD), _f32),
            pltpu.VMEM_SHARED((ACC_ROWS, HD), _f32),
            pltpu.SemaphoreType.DMA,
        ],
        interpret=interpret,
    )


_deg_kernel = _make_deg()
_prop_kernel = _make_prop()


def _tc_a_body(hist_ref, x_ref, w_ref, gA_ref, gB_ref, dinv_ref):
    deg = jnp.sum(hist_ref[...], axis=0)[:N] + 1.0
    dinv = lax.rsqrt(deg)
    dinv_ref[...] = dinv
    h = jnp.dot(x_ref[...], w_ref[...], preferred_element_type=_f32)
    g = h * dinv[:, None]
    gA_ref[...] = g[:, :HD]
    gB_ref[...] = g[:, HD:]


def _tc_b_body(pA_ref, pB_ref, gA_ref, gB_ref, dinv_ref, w_ref, b_ref,
               g2A_ref, g2B_ref):
    dinv = dinv_ref[...]
    sL = pA_ref[0:N] + pA_ref[ACC_ROWS:ACC_ROWS + N] + gA_ref[...]
    sR = pB_ref[0:N] + pB_ref[ACC_ROWS:ACC_ROWS + N] + gB_ref[...]
    b = b_ref[...]
    zL = jnp.maximum(sL * dinv[:, None] + b[:HD][None, :], 0.0)
    zR = jnp.maximum(sR * dinv[:, None] + b[HD:][None, :], 0.0)
    h2 = (jnp.dot(zL, w_ref[0:HD], preferred_element_type=_f32)
          + jnp.dot(zR, w_ref[HD:D], preferred_element_type=_f32))
    g2 = h2 * dinv[:, None]
    g2A_ref[...] = g2[:, :HD]
    g2B_ref[...] = g2[:, HD:]


def _tc_c_body(pA_ref, pB_ref, gA_ref, gB_ref, dinv_ref, b_ref, out_ref):
    dinv = dinv_ref[...]
    sL = pA_ref[0:N] + pA_ref[ACC_ROWS:ACC_ROWS + N] + gA_ref[...]
    sR = pB_ref[0:N] + pB_ref[ACC_ROWS:ACC_ROWS + N] + gB_ref[...]
    s = jnp.concatenate([sL, sR], axis=1)
    b = b_ref[...]
    z = s * dinv[:, None] + b[None, :]
    y = 1.0 / (1.0 + jnp.exp(-z))
    col = lax.broadcasted_iota(jnp.int32, (N, D), 1)
    out_ref[...] = jnp.where(col >= D - 5, jnp.float32(OWNER), y)


def _partition(idx, fill):
    counts = [(CPT_SLOW if (w // NS) != FAST_CID else CPT_FAST) * CHUNK
              for w in range(NW)]
    total = sum(counts)
    pad = jnp.full((total - E,), fill, jnp.int32)
    flat = jnp.concatenate([idx, pad])
    rows = []
    off = 0
    for w in range(NW):
        part = flat[off:off + counts[w]]
        off += counts[w]
        if counts[w] < CPT_FAST * CHUNK:
            part = jnp.concatenate(
                [part, jnp.full((CPT_FAST * CHUNK - counts[w],), fill,
                                jnp.int32)])
        rows.append(part.reshape(CPT_FAST, CHUNK))
    return jnp.stack(rows)


def kernel(x, edge_index, W1, b1, W2, b2):
    src = edge_index[0].astype(jnp.int32)
    dst = edge_index[1].astype(jnp.int32)
    dst_u = jnp.concatenate([dst, jnp.full((DEPAD - E,), DUMMY, jnp.int32)]
                            ).reshape(NW, DCPT, DCHUNK)
    src_t = _partition(src, 0)
    dst_t = _partition(dst, DUMMY)

    hist = _deg_kernel(dst_u).reshape(NW, ACC_ROWS)

    g1A, g1B, dinv = pl.pallas_call(
        _tc_a_body,
        out_shape=[jax.ShapeDtypeStruct((N, HD), _f32),
                   jax.ShapeDtypeStruct((N, HD), _f32),
                   jax.ShapeDtypeStruct((N,), _f32)],
    )(hist, x, W1)

    p1A, p1B = _prop_kernel(g1A, g1B, src_t, dst_t)

    g2A, g2B = pl.pallas_call(
        _tc_b_body,
        out_shape=[jax.ShapeDtypeStruct((N, HD), _f32),
                   jax.ShapeDtypeStruct((N, HD), _f32)],
    )(p1A, p1B, g1A, g1B, dinv, W2, b1)

    p2A, p2B = _prop_kernel(g2A, g2B, src_t, dst_t)

    out = pl.pallas_call(
        _tc_c_body,
        out_shape=jax.ShapeDtypeStruct((N, D), _f32),
    )(p2A, p2B, g2A, g2B, dinv, b2)

    return out


# speedup vs baseline: 1.0164x; 1.0164x over previous
import jax
import jax.numpy as jnp
from jax import lax
from jax.experimental import pallas as pl
from jax.experimental.pallas import tpu as pltpu
from jax.experimental.pallas import tpu_sc as plsc

N = 10000
D = 128
HD = D // 2
E = 320000
OWNER = 1.0

NC = 2
NS = 16
NW = NC * NS
CHUNK = 120
DCHUNK = 128
DCPT = -(-E // (NW * DCHUNK))
DEPAD = DCPT * DCHUNK * NW
FAST_CID = 0
CPT_SLOW = 72
CPT_FAST = 96
DUMMY = N
ACC_ROWS = 10112
ZROWS = ACC_ROWS // NS

_mesh = plsc.VectorSubcoreMesh(core_axis_name="c", subcore_axis_name="s",
                               num_cores=NC, num_subcores=NS)
_sc_params = pltpu.CompilerParams(
    needs_layout_passes=False, use_tc_tiling_on_sc=False)
_f32 = jnp.float32


def _deg_body(dst_hbm, out_hbm, dst_v, hist_v):
    wid = lax.axis_index("c") * NS + lax.axis_index("s")
    pltpu.sync_copy(dst_hbm.at[wid], dst_v)
    zeros = jnp.zeros((16,), _f32)

    def zbody(i, c):
        hist_v[pl.ds(i * 16, 16)] = zeros
        return c

    lax.fori_loop(0, ACC_ROWS // 16, zbody, 0)
    ones = jnp.ones((16,), _f32)

    def body(j, c):
        for k in range(DCHUNK // 16):
            idx = dst_v[j, pl.ds(k * 16, 16)]
            plsc.addupdate_scatter(hist_v, [idx], ones)
        return c

    lax.fori_loop(0, DCPT, body, 0)
    pltpu.sync_copy(hist_v, out_hbm.at[pl.ds(wid * ACC_ROWS, ACC_ROWS)])


def _prop_body(gA, gB, src_hbm, dst_hbm, outA, outB,
               src_v, dst_v, b0, acc, g0):
    cid = lax.axis_index("c")
    sid = lax.axis_index("s")
    wid = cid * NS + sid
    pltpu.sync_copy(src_hbm.at[wid], src_v)
    pltpu.sync_copy(dst_hbm.at[wid], dst_v)
    zeros = jnp.zeros((16,), _f32)
    zbase = sid * ZROWS
    nfull = ZROWS // CHUNK
    rem = ZROWS - nfull * CHUNK

    for p in range(2):
        g_hbm = gA if p == 0 else gB
        out_hbm = outA if p == 0 else outB

        def zrow(i, c):
            for k in range(HD // 16):
                b0[i, pl.ds(k * 16, 16)] = zeros
            return c

        lax.fori_loop(0, CHUNK, zrow, 0)
        for c in range(nfull):
            pltpu.sync_copy(b0, acc.at[pl.ds(zbase + c * CHUNK, CHUNK)])
        if rem:
            pltpu.sync_copy(b0.at[pl.ds(0, rem)],
                            acc.at[pl.ds(zbase + nfull * CHUNK, rem)])
        plsc.subcore_barrier()

        def body(j, c):
            desc = pltpu.make_async_copy(g_hbm.at[src_v.at[j]], b0, g0)
            desc.start()
            desc.wait()
            pltpu.sync_copy(b0, acc.at[dst_v.at[j]], add=True)
            return c

        lax.fori_loop(0, CPT_SLOW, body, 0)

        @pl.when(cid == FAST_CID)
        def _rest():
            lax.fori_loop(CPT_SLOW, CPT_FAST, body, 0)
        plsc.subcore_barrier()
        pltpu.sync_copy(acc.at[pl.ds(zbase, ZROWS)],
                        out_hbm.at[pl.ds(cid * ACC_ROWS + zbase, ZROWS)])
        if p == 0:
            plsc.subcore_barrier()


def _make_deg(interpret=False):
    return pl.kernel(
        _deg_body,
        out_type=jax.ShapeDtypeStruct((NW * ACC_ROWS,), _f32),
        mesh=_mesh,
        compiler_params=_sc_params,
        scratch_types=[
            pltpu.VMEM((DCPT, DCHUNK), jnp.int32),
            pltpu.VMEM((ACC_ROWS,), _f32),
        ],
        interpret=interpret,
    )


def _make_prop(interpret=False):
    return pl.kernel(
        _prop_body,
        out_type=(jax.ShapeDtypeStruct((NC * ACC_ROWS, HD), _f32),
                  jax.ShapeDtypeStruct((NC * ACC_ROWS, HD), _f32)),
        mesh=_mesh,
        compiler_params=_sc_params,
        scratch_types=[
            pltpu.VMEM((CPT_FAST, CHUNK), jnp.int32),
            pltpu.VMEM((CPT_FAST, CHUNK), jnp.int32),
            pltpu.VMEM((CHUNK, HD), _f32),
            pltpu.VMEM_SHARED((ACC_ROWS, HD), _f32),
            pltpu.SemaphoreType.DMA,
        ],
        interpret=interpret,
    )


_deg_kernel = _make_deg()
_prop_kernel = _make_prop()


def _tc_a_body(hist_ref, x_ref, w_ref, gA_ref, gB_ref, dinv_ref):
    deg = jnp.sum(hist_ref[...], axis=0)[:N] + 1.0
    dinv = lax.rsqrt(deg)
    dinv_ref[...] = dinv
    h = jnp.dot(x_ref[...], w_ref[...], preferred_element_type=_f32)
    g = h * dinv[:, None]
    gA_ref[...] = g[:, :HD]
    gB_ref[...] = g[:, HD:]


def _tc_b_body(pA_ref, pB_ref, gA_ref, gB_ref, dinv_ref, w_ref, b_ref,
               g2A_ref, g2B_ref):
    dinv = dinv_ref[...]
    sL = pA_ref[0:N] + pA_ref[ACC_ROWS:ACC_ROWS + N] + gA_ref[...]
    sR = pB_ref[0:N] + pB_ref[ACC_ROWS:ACC_ROWS + N] + gB_ref[...]
    b = b_ref[...]
    zL = jnp.maximum(sL * dinv[:, None] + b[:HD][None, :], 0.0)
    zR = jnp.maximum(sR * dinv[:, None] + b[HD:][None, :], 0.0)
    h2 = (jnp.dot(zL, w_ref[0:HD], preferred_element_type=_f32)
          + jnp.dot(zR, w_ref[HD:D], preferred_element_type=_f32))
    g2 = h2 * dinv[:, None]
    g2A_ref[...] = g2[:, :HD]
    g2B_ref[...] = g2[:, HD:]


def _tc_c_body(pA_ref, pB_ref, gA_ref, gB_ref, dinv_ref, b_ref, out_ref):
    dinv = dinv_ref[...]
    sL = pA_ref[0:N] + pA_ref[ACC_ROWS:ACC_ROWS + N] + gA_ref[...]
    sR = pB_ref[0:N] + pB_ref[ACC_ROWS:ACC_ROWS + N] + gB_ref[...]
    s = jnp.concatenate([sL, sR], axis=1)
    b = b_ref[...]
    z = s * dinv[:, None] + b[None, :]
    y = 1.0 / (1.0 + jnp.exp(-z))
    col = lax.broadcasted_iota(jnp.int32, (N, D), 1)
    out_ref[...] = jnp.where(col >= D - 5, jnp.float32(OWNER), y)


def _partition(idx, fill):
    counts = [(CPT_SLOW if (w // NS) != FAST_CID else CPT_FAST) * CHUNK
              for w in range(NW)]
    total = sum(counts)
    pad = jnp.full((total - E,), fill, jnp.int32)
    flat = jnp.concatenate([idx, pad])
    rows = []
    off = 0
    for w in range(NW):
        part = flat[off:off + counts[w]]
        off += counts[w]
        if counts[w] < CPT_FAST * CHUNK:
            part = jnp.concatenate(
                [part, jnp.full((CPT_FAST * CHUNK - counts[w],), fill,
                                jnp.int32)])
        rows.append(part.reshape(CPT_FAST, CHUNK))
    return jnp.stack(rows)


def kernel(x, edge_index, W1, b1, W2, b2):
    src = edge_index[0].astype(jnp.int32)
    dst = edge_index[1].astype(jnp.int32)
    dst_u = jnp.concatenate([dst, jnp.full((DEPAD - E,), DUMMY, jnp.int32)]
                            ).reshape(NW, DCPT, DCHUNK)
    src_t = _partition(src, 0)
    dst_t = _partition(dst, DUMMY)

    hist = _deg_kernel(dst_u).reshape(NW, ACC_ROWS)

    g1A, g1B, dinv = pl.pallas_call(
        _tc_a_body,
        out_shape=[jax.ShapeDtypeStruct((N, HD), _f32),
                   jax.ShapeDtypeStruct((N, HD), _f32),
                   jax.ShapeDtypeStruct((N,), _f32)],
    )(hist, x, W1)

    p1A, p1B = _prop_kernel(g1A, g1B, src_t, dst_t)

    g2A, g2B = pl.pallas_call(
        _tc_b_body,
        out_shape=[jax.ShapeDtypeStruct((N, HD), _f32),
                   jax.ShapeDtypeStruct((N, HD), _f32)],
    )(p1A, p1B, g1A, g1B, dinv, W2, b1)

    p2A, p2B = _prop_kernel(g2A, g2B, src_t, dst_t)

    out = pl.pallas_call(
        _tc_c_body,
        out_shape=jax.ShapeDtypeStruct((N, D), _f32),
    )(p2A, p2B, g2A, g2B, dinv, b2)

    return out
